# baseline (device time: 84546 ns/iter reference)
import jax
import jax.numpy as jnp
from jax import lax
from jax.experimental import pallas as pl
from jax.experimental.pallas import tpu as pltpu

N_EXP = 8
E_LOCAL = 4
E_HALF = 2
C = 320


def kernel(x, assign, W1, W2):
    m, d = x.shape
    f = W1.shape[-1]
    sc = E_LOCAL * C
    hc = E_HALF * C

    my_x = lax.axis_index("x")
    my_z = lax.axis_index("z")
    xb = x.astype(jnp.bfloat16)

    assign = assign.astype(jnp.int32)
    onehot = (jnp.arange(N_EXP, dtype=jnp.int32)[:, None] == assign[None, :])
    onehot = onehot.astype(jnp.int32)
    csum = jnp.cumsum(onehot, axis=1)
    slot = jnp.sum(csum * onehot, axis=0) - 1
    a_val = (onehot * (jnp.arange(m, dtype=jnp.int32) + 1)[None, :]).astype(
        jnp.float32
    )
    s_oh = (slot[:, None] == jnp.arange(C, dtype=jnp.int32)[None, :]).astype(
        jnp.float32
    )
    val = jnp.dot(a_val, s_oh, precision=lax.Precision.HIGHEST)
    idx = jnp.where(val == 0.0, m, val.astype(jnp.int32) - 1)

    def side_idx(side_x):
        full = lax.dynamic_slice_in_dim(idx, side_x * E_LOCAL, E_LOCAL, 0)
        mine = lax.dynamic_slice_in_dim(full, my_z * E_HALF, E_HALF, 0)
        other = lax.dynamic_slice_in_dim(full, (1 - my_z) * E_HALF, E_HALF, 0)
        return jnp.concatenate([mine, other], axis=0)

    tok = jnp.arange(m, dtype=jnp.int32)[None, :]
    P_l = (side_idx(my_x).reshape(sc, 1) == tok).astype(jnp.bfloat16)
    P_s = (side_idx(1 - my_x).reshape(sc, 1) == tok).astype(jnp.bfloat16)

    def body(
        xb_ref, pl_ref, ps_ref, w1_ref, w2_ref, out_ref,
        xs_ref, xr_ref, yb_ref, yl_ref, yr_ref,
        w1buf_ref, w2buf_ref, s1_ref, s2_ref,
        dsend, drecv, rsend, rrecv, gsend, grecv, zsend, zrecv,
        w1_sems, w2_sems,
    ):
        me = lax.axis_index("x")
        my = lax.axis_index("y")
        mz = lax.axis_index("z")
        xpeer = (1 - me, my, mz)
        zpeer = (me, my, 1 - mz)
        diag = (1 - me, my, 1 - mz)

        barrier = pltpu.get_barrier_semaphore()
        for nbr in (xpeer, zpeer, diag):
            pl.semaphore_signal(
                barrier, inc=1, device_id=nbr,
                device_id_type=pl.DeviceIdType.MESH,
            )
        pl.semaphore_wait(barrier, 3)

        def w_copies(j):
            ez = mz * E_HALF + j
            return (
                pltpu.make_async_copy(w1_ref.at[ez], s1_ref, w1_sems.at[0]),
                pltpu.make_async_copy(w2_ref.at[ez], s2_ref, w2_sems.at[0]),
            )

        for cp in w_copies(0):
            cp.start()

        rx = []
        for j in range(E_HALF):
            rows = pl.ds(j * C, C)
            xs_ref[rows, :] = jnp.dot(
                ps_ref[rows, :], xb_ref[...],
                preferred_element_type=jnp.float32,
            ).astype(jnp.bfloat16)
            r = pltpu.make_async_remote_copy(
                src_ref=xs_ref.at[rows], dst_ref=xr_ref.at[rows],
                send_sem=dsend.at[j], recv_sem=drecv.at[j],
                device_id=xpeer, device_id_type=pl.DeviceIdType.MESH,
            )
            r.start()
            rx.append(r)

        def ffn(slab):
            h = jnp.maximum(
                jnp.dot(slab, w1buf_ref[...],
                        preferred_element_type=jnp.float32),
                0.0,
            ).astype(jnp.bfloat16)
            return jnp.dot(
                h, w2buf_ref[...], preferred_element_type=jnp.float32
            ).astype(jnp.bfloat16)

        zwaits = []
        xwaits = []
        for j in range(E_HALF):
            for cp in w_copies(j):
                cp.wait()
            w1buf_ref[...] = s1_ref[...].astype(jnp.bfloat16)
            w2buf_ref[...] = s2_ref[...].astype(jnp.bfloat16)
            if j + 1 < E_HALF:
                for cp in w_copies(j + 1):
                    cp.start()
            rows = pl.ds(j * C, C)
            rows_hi = pl.ds((E_HALF + j) * C, C)

            slab_l = jnp.dot(
                pl_ref[rows, :], xb_ref[...],
                preferred_element_type=jnp.float32,
            ).astype(jnp.bfloat16)
            yl_ref[rows, :] = ffn(slab_l)
            rz = pltpu.make_async_remote_copy(
                src_ref=yl_ref.at[rows], dst_ref=yl_ref.at[rows_hi],
                send_sem=zsend.at[j], recv_sem=zrecv.at[j],
                device_id=zpeer, device_id_type=pl.DeviceIdType.MESH,
            )
            rz.start()
            zwaits.append(rz)

            rx[j].wait()
            yb_ref[rows, :] = ffn(xr_ref[rows, :])
            rr = pltpu.make_async_remote_copy(
                src_ref=yb_ref.at[rows], dst_ref=yr_ref.at[rows],
                send_sem=rsend.at[j], recv_sem=rrecv.at[j],
                device_id=xpeer, device_id_type=pl.DeviceIdType.MESH,
            )
            rr.start()
            xwaits.append(rr)
            rg = pltpu.make_async_remote_copy(
                src_ref=yb_ref.at[rows], dst_ref=yr_ref.at[rows_hi],
                send_sem=gsend.at[j], recv_sem=grecv.at[j],
                device_id=diag, device_id_type=pl.DeviceIdType.MESH,
            )
            rg.start()
            xwaits.append(rg)

        TC = 512
        for r in zwaits:
            r.wait()
        for cix in range(m // TC):
            cols = pl.ds(cix * TC, TC)
            loc = lax.dot_general(
                pl_ref[:, cols], yl_ref[...],
                dimension_numbers=(((0,), (0,)), ((), ())),
                preferred_element_type=jnp.float32,
            )
            out_ref[cols, :] = loc.astype(jnp.bfloat16)
        for r in xwaits:
            r.wait()
        for cix in range(m // TC):
            cols = pl.ds(cix * TC, TC)
            rem = lax.dot_general(
                ps_ref[:, cols], yr_ref[...],
                dimension_numbers=(((0,), (0,)), ((), ())),
                preferred_element_type=jnp.float32,
            )
            out_ref[cols, :] = out_ref[cols, :] + rem.astype(jnp.bfloat16)

    out = pl.pallas_call(
        body,
        out_shape=jax.ShapeDtypeStruct((m, d), jnp.bfloat16),
        in_specs=[
            pl.BlockSpec(memory_space=pltpu.VMEM),
            pl.BlockSpec(memory_space=pltpu.VMEM),
            pl.BlockSpec(memory_space=pltpu.VMEM),
            pl.BlockSpec(memory_space=pl.ANY),
            pl.BlockSpec(memory_space=pl.ANY),
        ],
        out_specs=pl.BlockSpec(memory_space=pltpu.VMEM),
        scratch_shapes=[
            pltpu.VMEM((hc, d), jnp.bfloat16),
            pltpu.VMEM((hc, d), jnp.bfloat16),
            pltpu.VMEM((hc, d), jnp.bfloat16),
            pltpu.VMEM((sc, d), jnp.bfloat16),
            pltpu.VMEM((sc, d), jnp.bfloat16),
            pltpu.VMEM((d, f), jnp.bfloat16),
            pltpu.VMEM((f, d), jnp.bfloat16),
            pltpu.VMEM((d, f), jnp.float32),
            pltpu.VMEM((f, d), jnp.float32),
            pltpu.SemaphoreType.DMA((E_HALF,)),
            pltpu.SemaphoreType.DMA((E_HALF,)),
            pltpu.SemaphoreType.DMA((E_HALF,)),
            pltpu.SemaphoreType.DMA((E_HALF,)),
            pltpu.SemaphoreType.DMA((E_HALF,)),
            pltpu.SemaphoreType.DMA((E_HALF,)),
            pltpu.SemaphoreType.DMA((E_HALF,)),
            pltpu.SemaphoreType.DMA((E_HALF,)),
            pltpu.SemaphoreType.DMA((1,)),
            pltpu.SemaphoreType.DMA((1,)),
        ],
        compiler_params=pltpu.CompilerParams(
            collective_id=0,
            vmem_limit_bytes=100 * 1024 * 1024,
        ),
    )(xb, P_l, P_s, W1, W2)
    return out


# device time: 81242 ns/iter; 1.0407x vs baseline; 1.0407x over previous
import jax
import jax.numpy as jnp
from jax import lax
from jax.experimental import pallas as pl
from jax.experimental.pallas import tpu as pltpu

N_EXP = 8
E_LOCAL = 4
E_HALF = 2
C = 320


def kernel(x, assign, W1, W2):
    m, d = x.shape
    f = W1.shape[-1]
    sc = E_LOCAL * C
    hc = E_HALF * C

    my_x = lax.axis_index("x")
    my_z = lax.axis_index("z")

    assign = assign.astype(jnp.int32)
    onehot = (jnp.arange(N_EXP, dtype=jnp.int32)[:, None] == assign[None, :])
    onehot = onehot.astype(jnp.int32)
    csum = jnp.cumsum(onehot, axis=1)
    slot = jnp.sum(csum * onehot, axis=0) - 1
    a_val = (onehot * (jnp.arange(m, dtype=jnp.int32) + 1)[None, :]).astype(
        jnp.float32
    )
    s_oh = (slot[:, None] == jnp.arange(C, dtype=jnp.int32)[None, :]).astype(
        jnp.float32
    )
    val = jnp.dot(a_val, s_oh, precision=lax.Precision.HIGHEST)
    idx = jnp.where(val == 0.0, m, val.astype(jnp.int32) - 1)

    def side_idx(side_x):
        full = lax.dynamic_slice_in_dim(idx, side_x * E_LOCAL, E_LOCAL, 0)
        mine = lax.dynamic_slice_in_dim(full, my_z * E_HALF, E_HALF, 0)
        other = lax.dynamic_slice_in_dim(full, (1 - my_z) * E_HALF, E_HALF, 0)
        return jnp.concatenate([mine, other], axis=0)

    tok = jnp.arange(m, dtype=jnp.int32)[None, :]
    P_l = (side_idx(my_x).reshape(sc, 1) == tok).astype(jnp.bfloat16)
    P_s = (side_idx(1 - my_x).reshape(sc, 1) == tok).astype(jnp.bfloat16)

    def body(
        xf_ref, pl_ref, ps_ref, w1_ref, w2_ref, out_ref,
        xb_ref, xs_ref, xr_ref, yb_ref, yl_ref, yr_ref,
        w1buf_ref, w2buf_ref, s1_ref, s2_ref,
        dsend, drecv, rsend, rrecv, gsend, grecv, zsend, zrecv,
        w1_sems, w2_sems,
    ):
        me = lax.axis_index("x")
        my = lax.axis_index("y")
        mz = lax.axis_index("z")
        xpeer = (1 - me, my, mz)
        zpeer = (me, my, 1 - mz)
        diag = (1 - me, my, 1 - mz)

        for cix in range(4):
            rs = pl.ds(cix * (m // 4), m // 4)
            xb_ref[rs, :] = xf_ref[rs, :].astype(jnp.bfloat16)

        barrier = pltpu.get_barrier_semaphore()
        for nbr in (xpeer, zpeer, diag):
            pl.semaphore_signal(
                barrier, inc=1, device_id=nbr,
                device_id_type=pl.DeviceIdType.MESH,
            )
        pl.semaphore_wait(barrier, 3)

        def w_copies(j):
            ez = mz * E_HALF + j
            return (
                pltpu.make_async_copy(w1_ref.at[ez], s1_ref, w1_sems.at[0]),
                pltpu.make_async_copy(w2_ref.at[ez], s2_ref, w2_sems.at[0]),
            )

        for cp in w_copies(0):
            cp.start()

        rx = []
        for j in range(E_HALF):
            rows = pl.ds(j * C, C)
            xs_ref[rows, :] = jnp.dot(
                ps_ref[rows, :], xb_ref[...],
                preferred_element_type=jnp.float32,
            ).astype(jnp.bfloat16)
            r = pltpu.make_async_remote_copy(
                src_ref=xs_ref.at[rows], dst_ref=xr_ref.at[rows],
                send_sem=dsend.at[j], recv_sem=drecv.at[j],
                device_id=xpeer, device_id_type=pl.DeviceIdType.MESH,
            )
            r.start()
            rx.append(r)

        def ffn(slab):
            h = jnp.maximum(
                jnp.dot(slab, w1buf_ref[...],
                        preferred_element_type=jnp.float32),
                0.0,
            ).astype(jnp.bfloat16)
            return jnp.dot(
                h, w2buf_ref[...], preferred_element_type=jnp.float32
            ).astype(jnp.bfloat16)

        zwaits = []
        xwaits = []
        for j in range(E_HALF):
            for cp in w_copies(j):
                cp.wait()
            w1buf_ref[...] = s1_ref[...].astype(jnp.bfloat16)
            w2buf_ref[...] = s2_ref[...].astype(jnp.bfloat16)
            if j + 1 < E_HALF:
                for cp in w_copies(j + 1):
                    cp.start()
            rows = pl.ds(j * C, C)
            rows_hi = pl.ds((E_HALF + j) * C, C)

            slab_l = jnp.dot(
                pl_ref[rows, :], xb_ref[...],
                preferred_element_type=jnp.float32,
            ).astype(jnp.bfloat16)
            yl_ref[rows, :] = ffn(slab_l)
            rz = pltpu.make_async_remote_copy(
                src_ref=yl_ref.at[rows], dst_ref=yl_ref.at[rows_hi],
                send_sem=zsend.at[j], recv_sem=zrecv.at[j],
                device_id=zpeer, device_id_type=pl.DeviceIdType.MESH,
            )
            rz.start()
            zwaits.append(rz)

            rx[j].wait()
            yb_ref[rows, :] = ffn(xr_ref[rows, :])
            rr = pltpu.make_async_remote_copy(
                src_ref=yb_ref.at[rows], dst_ref=yr_ref.at[rows],
                send_sem=rsend.at[j], recv_sem=rrecv.at[j],
                device_id=xpeer, device_id_type=pl.DeviceIdType.MESH,
            )
            rr.start()
            xwaits.append(rr)
            rg = pltpu.make_async_remote_copy(
                src_ref=yb_ref.at[rows], dst_ref=yr_ref.at[rows_hi],
                send_sem=gsend.at[j], recv_sem=grecv.at[j],
                device_id=diag, device_id_type=pl.DeviceIdType.MESH,
            )
            rg.start()
            xwaits.append(rg)

        TC = 512
        for r in zwaits:
            r.wait()
        for cix in range(m // TC):
            cols = pl.ds(cix * TC, TC)
            loc = lax.dot_general(
                pl_ref[:, cols], yl_ref[...],
                dimension_numbers=(((0,), (0,)), ((), ())),
                preferred_element_type=jnp.float32,
            )
            out_ref[cols, :] = loc.astype(jnp.bfloat16)
        for r in xwaits:
            r.wait()
        for cix in range(m // TC):
            cols = pl.ds(cix * TC, TC)
            rem = lax.dot_general(
                ps_ref[:, cols], yr_ref[...],
                dimension_numbers=(((0,), (0,)), ((), ())),
                preferred_element_type=jnp.float32,
            )
            out_ref[cols, :] = out_ref[cols, :] + rem.astype(jnp.bfloat16)

    out = pl.pallas_call(
        body,
        out_shape=jax.ShapeDtypeStruct((m, d), jnp.bfloat16),
        in_specs=[
            pl.BlockSpec(memory_space=pltpu.VMEM),
            pl.BlockSpec(memory_space=pltpu.VMEM),
            pl.BlockSpec(memory_space=pltpu.VMEM),
            pl.BlockSpec(memory_space=pl.ANY),
            pl.BlockSpec(memory_space=pl.ANY),
        ],
        out_specs=pl.BlockSpec(memory_space=pltpu.VMEM),
        scratch_shapes=[
            pltpu.VMEM((m, d), jnp.bfloat16),
            pltpu.VMEM((hc, d), jnp.bfloat16),
            pltpu.VMEM((hc, d), jnp.bfloat16),
            pltpu.VMEM((hc, d), jnp.bfloat16),
            pltpu.VMEM((sc, d), jnp.bfloat16),
            pltpu.VMEM((sc, d), jnp.bfloat16),
            pltpu.VMEM((d, f), jnp.bfloat16),
            pltpu.VMEM((f, d), jnp.bfloat16),
            pltpu.VMEM((d, f), jnp.float32),
            pltpu.VMEM((f, d), jnp.float32),
            pltpu.SemaphoreType.DMA((E_HALF,)),
            pltpu.SemaphoreType.DMA((E_HALF,)),
            pltpu.SemaphoreType.DMA((E_HALF,)),
            pltpu.SemaphoreType.DMA((E_HALF,)),
            pltpu.SemaphoreType.DMA((E_HALF,)),
            pltpu.SemaphoreType.DMA((E_HALF,)),
            pltpu.SemaphoreType.DMA((E_HALF,)),
            pltpu.SemaphoreType.DMA((E_HALF,)),
            pltpu.SemaphoreType.DMA((1,)),
            pltpu.SemaphoreType.DMA((1,)),
        ],
        compiler_params=pltpu.CompilerParams(
            collective_id=0,
            vmem_limit_bytes=100 * 1024 * 1024,
        ),
    )(x, P_l, P_s, W1, W2)
    return out
